# baseline (speedup 1.0000x reference)
import functools

import jax
import jax.numpy as jnp
import numpy as np
from jax import lax
from jax.experimental import pallas as pl
from jax.experimental.pallas import tpu as pltpu
from jax.experimental.pallas import tpu_sc as plsc

NC = 1
NS = 16
NW = NC * NS
LANES = 16

B = 16384
BH = B // 2
L = 50
V = 5895
D = 100
DP = 128
NVREG = DP // 32
ROWS_PER_W = BH // NW
ROWS_PER_CHUNK = 2
CHUNK_IDX = ROWS_PER_CHUNK * L
NCHUNK = ROWS_PER_W // ROWS_PER_CHUNK
NBUF = 10
MBLK = 64
CHUNKS_PER_MBLK = MBLK // ROWS_PER_CHUNK


def _sc_pool_body(emb_hbm, idx_hbm, out_hbm, idx_v, bufs, sems, m_v):
    wid = lax.axis_index("s")

    pltpu.sync_copy(idx_hbm.at[wid], idx_v)

    def _start(chunk, slot):
        pltpu.make_async_copy(
            emb_hbm.at[idx_v.at[chunk]], bufs[slot], sems[slot]
        ).start()

    def _wait(slot):
        pltpu.make_async_copy(
            emb_hbm.at[idx_v.at[0]], bufs[slot], sems[slot]
        ).wait()

    for slot in range(NBUF):
        _start(slot, slot)

    outer = -(-NCHUNK // NBUF)

    def body(t, carry):
        for slot in range(NBUF):
            g = NBUF * t + slot

            @pl.when(g < NCHUNK)
            def _():
                _wait(slot)
                buf = bufs[slot]
                RUN = 5
                for j in range(ROWS_PER_CHUNK):
                    row = (ROWS_PER_CHUNK * g + j) & (MBLK - 1)

                    def racc(rb, accs, j=j, buf=buf):
                        lo, hi = accs
                        lo = list(lo)
                        hi = list(hi)
                        for rr in range(RUN):
                            r = j * L + rb * RUN + rr
                            for v in range(NVREG):
                                ab = buf[r, pl.ds(v * 32, 32)]
                                a, b = plsc.unpack(
                                    ab,
                                    format=plsc.PackFormat.INTERLEAVED)
                                lo[v] = lo[v] + a
                                hi[v] = hi[v] + b
                        return (tuple(lo), tuple(hi))

                    zero = jnp.zeros((LANES,), jnp.float32)
                    lo, hi = lax.fori_loop(
                        0, L // RUN, racc,
                        ((zero,) * NVREG, (zero,) * NVREG))
                    for v in range(NVREG):
                        m_v[row, pl.ds(32 * v, LANES)] = lo[v]
                        m_v[row, pl.ds(32 * v + LANES, LANES)] = hi[v]
                nxt = g + NBUF

                @pl.when(nxt < NCHUNK)
                def _():
                    _start(nxt, slot)

                @pl.when((g + 1) % CHUNKS_PER_MBLK == 0)
                def _():
                    base = (wid * ROWS_PER_W
                            + (g + 1 - CHUNKS_PER_MBLK) * ROWS_PER_CHUNK)
                    base = pl.multiple_of(base, MBLK)
                    pltpu.sync_copy(m_v, out_hbm.at[pl.ds(base, MBLK)])

        return carry

    lax.fori_loop(0, outer, body, 0)


_sc_pool = functools.partial(
    pl.kernel,
    out_type=jax.ShapeDtypeStruct((BH, DP), jnp.float32),
    compiler_params=pltpu.CompilerParams(
        use_tc_tiling_on_sc=False, needs_layout_passes=False),
    mesh=plsc.VectorSubcoreMesh(core_axis_name="c", subcore_axis_name="s",
                                num_cores=1),
    scratch_types=[
        pltpu.VMEM((NCHUNK, CHUNK_IDX), jnp.int32),
        [pltpu.VMEM((CHUNK_IDX, DP), jnp.bfloat16) for _ in range(NBUF)],
        [pltpu.SemaphoreType.DMA for _ in range(NBUF)],
        pltpu.VMEM((MBLK, DP), jnp.float32),
    ],
)(_sc_pool_body)


MLP_BLK = 2048
HID = 384
OUTP = 128

_PERM = np.concatenate(
    [np.concatenate([np.arange(32 * c, 32 * c + 32, 2),
                     np.arange(32 * c + 1, 32 * c + 32, 2)])
     for c in range(DP // 32)]
)


def _mlp_body(m_ref, w1_ref, b1_ref, w2_ref, b2_ref, o_ref):
    h = jnp.dot(m_ref[...], w1_ref[...], preferred_element_type=jnp.float32)
    h = jnp.maximum(h + b1_ref[...], 0.0)
    o = jnp.dot(h, w2_ref[...], preferred_element_type=jnp.float32)
    o_ref[...] = o + b2_ref[...]


def _mlp(m, w1, b1, w2, b2):
    grid = (B // MLP_BLK,)
    return pl.pallas_call(
        _mlp_body,
        grid=grid,
        in_specs=[
            pl.BlockSpec((MLP_BLK, DP), lambda i: (i, 0)),
            pl.BlockSpec((DP, HID), lambda i: (0, 0)),
            pl.BlockSpec((1, HID), lambda i: (0, 0)),
            pl.BlockSpec((HID, OUTP), lambda i: (0, 0)),
            pl.BlockSpec((1, OUTP), lambda i: (0, 0)),
        ],
        out_specs=pl.BlockSpec((MLP_BLK, OUTP), lambda i: (i, 0)),
        out_shape=jax.ShapeDtypeStruct((B, OUTP), jnp.float32),
    )(m, w1, b1, w2, b2)


@jax.jit
def kernel(x, emb, W1, b1, W2, b2):
    emb_bf = jnp.pad(emb, ((0, 0), (0, DP - D))).astype(jnp.bfloat16)
    idx = x.astype(jnp.int32).reshape(2, NW, NCHUNK, CHUNK_IDX)
    w1 = jnp.pad(W1 * (1.0 / L), ((0, DP - D), (0, HID - 300)))[_PERM]
    b1p = jnp.pad(b1, (0, HID - 300)).reshape(1, HID)
    w2 = jnp.pad(W2, ((0, HID - 300), (0, OUTP - 3)))
    b2p = jnp.pad(b2, (0, OUTP - 3)).reshape(1, OUTP)

    m0 = _sc_pool(emb_bf, idx[0])
    m1 = _sc_pool(emb_bf, idx[1])
    m = jnp.concatenate([m0, m1], axis=0)
    out = _mlp(m, w1, b1p, w2, b2p)
    return out[:, :3]


# speedup vs baseline: 1.4772x; 1.4772x over previous
import functools

import jax
import jax.numpy as jnp
import numpy as np
from jax import lax
from jax.experimental import pallas as pl
from jax.experimental.pallas import tpu as pltpu
from jax.experimental.pallas import tpu_sc as plsc

NC = 2
NS = 16
NW = NC * NS
LANES = 16

B = 16384
L = 50
V = 5895
D = 100
DP = 128
NVREG = DP // 32
ROWS_PER_W = B // NW
ROWS_PER_CHUNK = 2
CHUNK_IDX = ROWS_PER_CHUNK * L
NCHUNK = ROWS_PER_W // ROWS_PER_CHUNK
NBUF = 10
MBLK = 64
CHUNKS_PER_MBLK = MBLK // ROWS_PER_CHUNK


def _sc_pool_body(emb_hbm, idx_hbm, out_hbm, idx_v, bufs, sems, m_v):
    wid = lax.axis_index("s") * NC + lax.axis_index("c")

    pltpu.sync_copy(idx_hbm.at[wid], idx_v)

    def _start(chunk, slot):
        pltpu.make_async_copy(
            emb_hbm.at[idx_v.at[chunk]], bufs[slot], sems[slot]
        ).start()

    def _wait(slot):
        pltpu.make_async_copy(
            emb_hbm.at[idx_v.at[0]], bufs[slot], sems[slot]
        ).wait()

    for slot in range(NBUF):
        _start(slot, slot)

    outer = -(-NCHUNK // NBUF)

    def body(t, carry):
        for slot in range(NBUF):
            g = NBUF * t + slot

            @pl.when(g < NCHUNK)
            def _():
                _wait(slot)
                buf = bufs[slot]
                RUN = 5
                for j in range(ROWS_PER_CHUNK):
                    row = (ROWS_PER_CHUNK * g + j) & (MBLK - 1)

                    def racc(rb, accs, j=j, buf=buf):
                        lo, hi = accs
                        lo = list(lo)
                        hi = list(hi)
                        for rr in range(RUN):
                            r = j * L + rb * RUN + rr
                            for v in range(NVREG):
                                ab = buf[r, pl.ds(v * 32, 32)]
                                a, b = plsc.unpack(
                                    ab,
                                    format=plsc.PackFormat.INTERLEAVED)
                                lo[v] = lo[v] + a
                                hi[v] = hi[v] + b
                        return (tuple(lo), tuple(hi))

                    zero = jnp.zeros((LANES,), jnp.float32)
                    lo, hi = lax.fori_loop(
                        0, L // RUN, racc,
                        ((zero,) * NVREG, (zero,) * NVREG))
                    for v in range(NVREG):
                        m_v[row, pl.ds(32 * v, 32)] = plsc.pack(
                            lo[v], hi[v],
                            format=plsc.PackFormat.INTERLEAVED)
                nxt = g + NBUF

                @pl.when(nxt < NCHUNK)
                def _():
                    _start(nxt, slot)

                @pl.when((g + 1) % CHUNKS_PER_MBLK == 0)
                def _():
                    base = (wid * ROWS_PER_W
                            + (g + 1 - CHUNKS_PER_MBLK) * ROWS_PER_CHUNK)
                    base = pl.multiple_of(base, MBLK)
                    pltpu.sync_copy(m_v, out_hbm.at[pl.ds(base, MBLK)])

        return carry

    lax.fori_loop(0, outer, body, 0)


_sc_pool = functools.partial(
    pl.kernel,
    out_type=jax.ShapeDtypeStruct((B, DP), jnp.bfloat16),
    compiler_params=pltpu.CompilerParams(
        use_tc_tiling_on_sc=False, needs_layout_passes=False),
    mesh=plsc.VectorSubcoreMesh(core_axis_name="c", subcore_axis_name="s"),
    scratch_types=[
        pltpu.VMEM((NCHUNK, CHUNK_IDX), jnp.int32),
        [pltpu.VMEM((CHUNK_IDX, DP), jnp.bfloat16) for _ in range(NBUF)],
        [pltpu.SemaphoreType.DMA for _ in range(NBUF)],
        pltpu.VMEM((MBLK, DP), jnp.bfloat16),
    ],
)(_sc_pool_body)


MLP_BLK = 2048
HID = 384
OUTP = 128

def _mlp_body(m_ref, w1_ref, b1_ref, w2_ref, b2_ref, o_ref):
    mf = m_ref[...].astype(jnp.float32)
    h = jnp.dot(mf, w1_ref[...], preferred_element_type=jnp.float32)
    h = jnp.maximum(h + b1_ref[...], 0.0)
    o = jnp.dot(h, w2_ref[...], preferred_element_type=jnp.float32)
    o_ref[...] = o + b2_ref[...]


def _mlp(m, w1, b1, w2, b2):
    grid = (B // MLP_BLK,)
    return pl.pallas_call(
        _mlp_body,
        grid=grid,
        in_specs=[
            pl.BlockSpec((MLP_BLK, DP), lambda i: (i, 0)),
            pl.BlockSpec((DP, HID), lambda i: (0, 0)),
            pl.BlockSpec((1, HID), lambda i: (0, 0)),
            pl.BlockSpec((HID, OUTP), lambda i: (0, 0)),
            pl.BlockSpec((1, OUTP), lambda i: (0, 0)),
        ],
        out_specs=pl.BlockSpec((MLP_BLK, OUTP), lambda i: (i, 0)),
        out_shape=jax.ShapeDtypeStruct((B, OUTP), jnp.float32),
    )(m, w1, b1, w2, b2)


@jax.jit
def kernel(x, emb, W1, b1, W2, b2):
    emb_bf = jnp.pad(emb, ((0, 0), (0, DP - D))).astype(jnp.bfloat16)
    idx = x.astype(jnp.int32).reshape(NW, NCHUNK, CHUNK_IDX)
    w1 = jnp.pad(W1 * (1.0 / L), ((0, DP - D), (0, HID - 300)))
    b1p = jnp.pad(b1, (0, HID - 300)).reshape(1, HID)
    w2 = jnp.pad(W2, ((0, HID - 300), (0, OUTP - 3)))
    b2p = jnp.pad(b2, (0, OUTP - 3)).reshape(1, OUTP)

    m = _sc_pool(emb_bf, idx)
    out = _mlp(m, w1, b1p, w2, b2p)
    return out[:, :3]


# speedup vs baseline: 1.5982x; 1.0819x over previous
import functools

import jax
import jax.numpy as jnp
import numpy as np
from jax import lax
from jax.experimental import pallas as pl
from jax.experimental.pallas import tpu as pltpu
from jax.experimental.pallas import tpu_sc as plsc

NC = 2
NS = 16
NW = NC * NS
LANES = 16

B = 16384
L = 50
V = 5895
D = 100
DP = 128
NVREG = DP // 32
ROWS_PER_W = B // NW
ROWS_PER_CHUNK = 2
CHUNK_IDX = ROWS_PER_CHUNK * L
NCHUNK = ROWS_PER_W // ROWS_PER_CHUNK
NBUF = 10
MBLK = 64
CHUNKS_PER_MBLK = MBLK // ROWS_PER_CHUNK


def _sc_pool_body(emb_hbm, idx_hbm, out_hbm, idx_v, bufs, sems, m_v):
    wid = lax.axis_index("s") * NC + lax.axis_index("c")

    pltpu.sync_copy(idx_hbm.at[wid], idx_v)

    def _start(chunk, slot):
        pltpu.make_async_copy(
            emb_hbm.at[idx_v.at[chunk]], bufs[slot], sems[slot]
        ).start()

    def _wait(slot):
        pltpu.make_async_copy(
            emb_hbm.at[idx_v.at[0]], bufs[slot], sems[slot]
        ).wait()

    for slot in range(NBUF):
        _start(slot, slot)

    outer = -(-NCHUNK // NBUF)

    def body(t, carry):
        for slot in range(NBUF):
            g = NBUF * t + slot

            @pl.when(g < NCHUNK)
            def _():
                _wait(slot)
                buf = bufs[slot]
                RUN = 5
                for j in range(ROWS_PER_CHUNK):
                    row = (ROWS_PER_CHUNK * g + j) & (MBLK - 1)

                    def racc(rb, accs, j=j, buf=buf):
                        lo, hi = accs
                        lo = list(lo)
                        hi = list(hi)
                        for rr in range(RUN):
                            r = j * L + rb * RUN + rr
                            for v in range(NVREG):
                                ab = buf[r, pl.ds(v * 32, 32)]
                                a, b = plsc.unpack(
                                    ab,
                                    format=plsc.PackFormat.INTERLEAVED)
                                lo[v] = lo[v] + a
                                hi[v] = hi[v] + b
                        return (tuple(lo), tuple(hi))

                    zero = jnp.zeros((LANES,), jnp.float32)
                    lo, hi = lax.fori_loop(
                        0, L // RUN, racc,
                        ((zero,) * NVREG, (zero,) * NVREG))
                    for v in range(NVREG):
                        m_v[row, pl.ds(32 * v, LANES)] = lo[v]
                        m_v[row, pl.ds(32 * v + LANES, LANES)] = hi[v]
                nxt = g + NBUF

                @pl.when(nxt < NCHUNK)
                def _():
                    _start(nxt, slot)

                @pl.when((g + 1) % CHUNKS_PER_MBLK == 0)
                def _():
                    base = (wid * ROWS_PER_W
                            + (g + 1 - CHUNKS_PER_MBLK) * ROWS_PER_CHUNK)
                    base = pl.multiple_of(base, MBLK)
                    pltpu.sync_copy(m_v, out_hbm.at[pl.ds(base, MBLK)])

        return carry

    lax.fori_loop(0, outer, body, 0)


_sc_pool = functools.partial(
    pl.kernel,
    out_type=jax.ShapeDtypeStruct((B, DP), jnp.float32),
    compiler_params=pltpu.CompilerParams(
        use_tc_tiling_on_sc=False, needs_layout_passes=False),
    mesh=plsc.VectorSubcoreMesh(core_axis_name="c", subcore_axis_name="s"),
    scratch_types=[
        pltpu.VMEM((NCHUNK, CHUNK_IDX), jnp.int32),
        [pltpu.VMEM((CHUNK_IDX, DP), jnp.bfloat16) for _ in range(NBUF)],
        [pltpu.SemaphoreType.DMA for _ in range(NBUF)],
        pltpu.VMEM((MBLK, DP), jnp.float32),
    ],
)(_sc_pool_body)


MLP_BLK = 2048
HID = 384
OUTP = 128

_PERM = np.concatenate(
    [np.concatenate([np.arange(32 * c, 32 * c + 32, 2),
                     np.arange(32 * c + 1, 32 * c + 32, 2)])
     for c in range(DP // 32)]
)


def _mlp_body(m_ref, w1_ref, b1_ref, w2_ref, b2_ref, o_ref):
    h = jnp.dot(m_ref[...], w1_ref[...], preferred_element_type=jnp.float32)
    h = jnp.maximum(h + b1_ref[...], 0.0)
    o = jnp.dot(h, w2_ref[...], preferred_element_type=jnp.float32)
    o_ref[...] = o + b2_ref[...]


def _mlp(m, w1, b1, w2, b2):
    grid = (B // MLP_BLK,)
    return pl.pallas_call(
        _mlp_body,
        grid=grid,
        in_specs=[
            pl.BlockSpec((MLP_BLK, DP), lambda i: (i, 0)),
            pl.BlockSpec((DP, HID), lambda i: (0, 0)),
            pl.BlockSpec((1, HID), lambda i: (0, 0)),
            pl.BlockSpec((HID, OUTP), lambda i: (0, 0)),
            pl.BlockSpec((1, OUTP), lambda i: (0, 0)),
        ],
        out_specs=pl.BlockSpec((MLP_BLK, OUTP), lambda i: (i, 0)),
        out_shape=jax.ShapeDtypeStruct((B, OUTP), jnp.float32),
    )(m, w1, b1, w2, b2)


@jax.jit
def kernel(x, emb, W1, b1, W2, b2):
    emb_bf = jnp.pad(emb, ((0, 0), (0, DP - D))).astype(jnp.bfloat16)
    idx = x.astype(jnp.int32).reshape(NW, NCHUNK, CHUNK_IDX)
    w1 = jnp.pad(W1 * (1.0 / L), ((0, DP - D), (0, HID - 300)))[_PERM]
    b1p = jnp.pad(b1, (0, HID - 300)).reshape(1, HID)
    w2 = jnp.pad(W2, ((0, HID - 300), (0, OUTP - 3)))
    b2p = jnp.pad(b2, (0, OUTP - 3)).reshape(1, OUTP)

    m = _sc_pool(emb_bf, idx)
    out = _mlp(m, w1, b1p, w2, b2p)
    return out[:, :3]


# speedup vs baseline: 1.6175x; 1.0121x over previous
import functools

import jax
import jax.numpy as jnp
import numpy as np
from jax import lax
from jax.experimental import pallas as pl
from jax.experimental.pallas import tpu as pltpu
from jax.experimental.pallas import tpu_sc as plsc

NC = 2
NS = 16
NW = NC * NS
LANES = 16

B = 16384
L = 50
V = 5895
D = 100
DP = 128
NVREG = DP // 32
ROWS_PER_W = B // NW
ROWS_PER_CHUNK = 2
CHUNK_IDX = ROWS_PER_CHUNK * L
NCHUNK = ROWS_PER_W // ROWS_PER_CHUNK
NBUF = 10
MBLK = 64
CHUNKS_PER_MBLK = MBLK // ROWS_PER_CHUNK


def _sc_pool_body(emb_hbm, idx_hbm, out_hbm, idx_v, bufs, sems, m_v):
    wid = lax.axis_index("s") * NC + lax.axis_index("c")

    pltpu.sync_copy(idx_hbm.at[wid], idx_v)

    def _start(chunk, slot):
        pltpu.make_async_copy(
            emb_hbm.at[idx_v.at[chunk]], bufs[slot], sems[slot]
        ).start()

    def _wait(slot):
        pltpu.make_async_copy(
            emb_hbm.at[idx_v.at[0]], bufs[slot], sems[slot]
        ).wait()

    for slot in range(NBUF):
        _start(slot, slot)

    outer = -(-NCHUNK // NBUF)

    def body(t, carry):
        for slot in range(NBUF):
            g = NBUF * t + slot

            @pl.when(g < NCHUNK)
            def _():
                _wait(slot)
                buf = bufs[slot]
                RUN = 5
                for j in range(ROWS_PER_CHUNK):
                    row = (ROWS_PER_CHUNK * g + j) & (MBLK - 1)

                    def racc(rb, accs, j=j, buf=buf):
                        lo, hi = accs
                        lo = list(lo)
                        hi = list(hi)
                        for rr in range(RUN):
                            r = j * L + rb * RUN + rr
                            for v in range(NVREG):
                                ab = buf[r, pl.ds(v * 32, 32)]
                                a, b = plsc.unpack(
                                    ab,
                                    format=plsc.PackFormat.INTERLEAVED)
                                lo[v] = lo[v] + a
                                hi[v] = hi[v] + b
                        return (tuple(lo), tuple(hi))

                    zero = jnp.zeros((LANES,), jnp.float32)
                    lo, hi = lax.fori_loop(
                        0, L // RUN, racc,
                        ((zero,) * NVREG, (zero,) * NVREG))
                    for v in range(NVREG):
                        m_v[row, pl.ds(32 * v, LANES)] = lo[v]
                        m_v[row, pl.ds(32 * v + LANES, LANES)] = hi[v]
                nxt = g + NBUF

                @pl.when(nxt < NCHUNK)
                def _():
                    _start(nxt, slot)

                @pl.when((g + 1) % CHUNKS_PER_MBLK == 0)
                def _():
                    base = (wid * ROWS_PER_W
                            + (g + 1 - CHUNKS_PER_MBLK) * ROWS_PER_CHUNK)
                    base = pl.multiple_of(base, MBLK)
                    pltpu.sync_copy(m_v, out_hbm.at[pl.ds(base, MBLK)])

        return carry

    lax.fori_loop(0, outer, body, 0)


_sc_pool = functools.partial(
    pl.kernel,
    out_type=jax.ShapeDtypeStruct((B, DP), jnp.float32),
    compiler_params=pltpu.CompilerParams(
        use_tc_tiling_on_sc=False, needs_layout_passes=False),
    mesh=plsc.VectorSubcoreMesh(core_axis_name="c", subcore_axis_name="s"),
    scratch_types=[
        pltpu.VMEM((NCHUNK, CHUNK_IDX), jnp.int32),
        [pltpu.VMEM((CHUNK_IDX, DP), jnp.bfloat16) for _ in range(NBUF)],
        [pltpu.SemaphoreType.DMA for _ in range(NBUF)],
        pltpu.VMEM((MBLK, DP), jnp.float32),
    ],
)(_sc_pool_body)


MLP_BLK = 4096
HID = 384
OUTP = 128

_PERM = np.concatenate(
    [np.concatenate([np.arange(32 * c, 32 * c + 32, 2),
                     np.arange(32 * c + 1, 32 * c + 32, 2)])
     for c in range(DP // 32)]
)


def _mlp_body(m_ref, w1_ref, b1_ref, w2_ref, b2_ref, o_ref):
    h = jnp.dot(m_ref[...], w1_ref[...], preferred_element_type=jnp.float32)
    h = jnp.maximum(h + b1_ref[...], 0.0)
    o = jnp.dot(h, w2_ref[...], preferred_element_type=jnp.float32)
    o_ref[...] = o + b2_ref[...]


def _mlp(m, w1, b1, w2, b2):
    grid = (B // MLP_BLK,)
    return pl.pallas_call(
        _mlp_body,
        grid=grid,
        in_specs=[
            pl.BlockSpec((MLP_BLK, DP), lambda i: (i, 0)),
            pl.BlockSpec((DP, HID), lambda i: (0, 0)),
            pl.BlockSpec((1, HID), lambda i: (0, 0)),
            pl.BlockSpec((HID, OUTP), lambda i: (0, 0)),
            pl.BlockSpec((1, OUTP), lambda i: (0, 0)),
        ],
        out_specs=pl.BlockSpec((MLP_BLK, OUTP), lambda i: (i, 0)),
        out_shape=jax.ShapeDtypeStruct((B, OUTP), jnp.float32),
    )(m, w1, b1, w2, b2)


@jax.jit
def kernel(x, emb, W1, b1, W2, b2):
    emb_bf = jnp.pad(emb, ((0, 0), (0, DP - D))).astype(jnp.bfloat16)
    idx = x.astype(jnp.int32).reshape(NW, NCHUNK, CHUNK_IDX)
    w1 = jnp.pad(W1 * (1.0 / L), ((0, DP - D), (0, HID - 300)))[_PERM]
    b1p = jnp.pad(b1, (0, HID - 300)).reshape(1, HID)
    w2 = jnp.pad(W2, ((0, HID - 300), (0, OUTP - 3)))
    b2p = jnp.pad(b2, (0, OUTP - 3)).reshape(1, OUTP)

    m = _sc_pool(emb_bf, idx)
    out = _mlp(m, w1, b1p, w2, b2p)
    return out[:, :3]
